# baseline (speedup 1.0000x reference)
import functools

import jax
import jax.numpy as jnp
from jax import lax
from jax.experimental import pallas as pl
from jax.experimental.pallas import tpu as pltpu
from jax.experimental.pallas import tpu_sc as plsc

_N = 20000
_K = 2000
_KP = 2048
_T = 128
_NT = _KP // _T
_TN = 256
_SCORE_THRESH = 0.05
_NMS_THRESH = 0.5
_DET = 100
_NEG = -1e9

_INTERPRET = False
_BW = 64


def _sc_gather_body(table_ref, idx_ref, t3_out_ref, idx_v, eidx_v, tstripe_v, sem):
    i32 = jnp.int32
    wid = lax.axis_index("s") * 2 + lax.axis_index("c")
    base = wid * _BW
    pltpu.sync_copy(idx_ref.at[pl.ds(base, _BW)], idx_v)
    for c in range(4):
        for u in range(4):
            bi = idx_v[u * 16:(u + 1) * 16]
            q = c * _BW + u * 16
            eidx_v[q // 128, (q % 128):(q % 128) + 16] = bi * 4 + c
    cp0 = pltpu.async_copy(table_ref.at[eidx_v.at[0]], tstripe_v.at[0], sem)
    cp1 = pltpu.async_copy(table_ref.at[eidx_v.at[1]], tstripe_v.at[1], sem)
    cp0.wait()
    cp1.wait()
    pltpu.sync_copy(tstripe_v, t3_out_ref.at[wid])


def _sc_gather(table, idx_p):
    mesh = plsc.VectorSubcoreMesh(core_axis_name="c", subcore_axis_name="s")
    run = functools.partial(
        pl.kernel,
        mesh=mesh,
        out_type=[jax.ShapeDtypeStruct((32, 2, 128), jnp.float32)],
        scratch_types=[pltpu.VMEM((_BW,), jnp.int32),
                       pltpu.VMEM((2, 128), jnp.int32),
                       pltpu.VMEM((2, 128), jnp.float32),
                       pltpu.SemaphoreType.DMA],
    )(_sc_gather_body)
    return run(table, idx_p)


def _nms_body(ts_ref, tb_ref, tbt_ref, out_ref, kscr_ref):
    f32 = jnp.float32
    tb = tb_ref[...]
    tbt = tbt_ref[...]
    x1c = tb[:, 0:1]
    y1c = tb[:, 1:2]
    x2c = tb[:, 2:3]
    y2c = tb[:, 3:4]
    x1r = tbt[0:1, :]
    y1r = tbt[1:2, :]
    x2r = tbt[2:3, :]
    y2r = tbt[3:4, :]
    ts = ts_ref[...]
    area_r = (x2r - x1r) * (y2r - y1r)
    area_c = (x2c - x1c) * (y2c - y1c)

    rows_tt = lax.broadcasted_iota(jnp.int32, (_T, _T), 0)
    cols_tt = lax.broadcasted_iota(jnp.int32, (_T, _T), 1)
    rows_nn = lax.broadcasted_iota(jnp.int32, (_TN, _TN), 0)
    cols_nn = lax.broadcasted_iota(jnp.int32, (_TN, _TN), 1)
    upper_f = (rows_nn < cols_nn).astype(f32)

    sup = jnp.zeros((1, _KP), jnp.bool_)
    keep_tiles = []
    for t in range(_KP // _TN):
        s = t * _TN
        tx1 = x1c[s:s + _TN]
        ty1 = y1c[s:s + _TN]
        tx2 = x2c[s:s + _TN]
        ty2 = y2c[s:s + _TN]
        ta = area_c[s:s + _TN]
        iw = jnp.clip(jnp.minimum(tx2, x2r[:, s:]) - jnp.maximum(tx1, x1r[:, s:]), 0.0)
        ih = jnp.clip(jnp.minimum(ty2, y2r[:, s:]) - jnp.maximum(ty1, y1r[:, s:]), 0.0)
        inter = iw * ih
        union = (ta + area_r[:, s:]) - inter + 1e-9
        mf = (inter > _NMS_THRESH * union).astype(f32)
        mtt = mf[:, :_TN] * upper_f
        sup_t = sup[:, s:s + _TN]
        availf = jnp.where(sup_t, 0.0, 1.0)
        kscr_ref[...] = availf

        def fbody(_, mtt=mtt, availf=availf):
            k = kscr_ref[...]
            s1 = jnp.dot(k, mtt, preferred_element_type=f32)
            k1 = availf * jnp.where(s1 < 0.5, 1.0, 0.0)
            s2 = jnp.dot(k1, mtt, preferred_element_type=f32)
            k2 = availf * jnp.where(s2 < 0.5, 1.0, 0.0)
            kscr_ref[...] = k2
            return jnp.any(k2 != k1)

        lax.while_loop(lambda c: c, fbody, jnp.bool_(True))
        kfin = kscr_ref[...] > 0.5
        keep_tiles.append(kfin)
        supnew = jnp.dot(kfin.astype(f32), mf, preferred_element_type=f32) > 0.5
        tail = sup[:, s:] | supnew
        sup = tail if s == 0 else jnp.concatenate([sup[:, :s], tail], axis=1)
    keep = jnp.concatenate(keep_tiles, axis=1)

    idx = lax.broadcasted_iota(jnp.int32, (1, _KP), 1)
    real = idx < _K
    kept_valid = keep & (ts > _NEG) & real
    other_real = real & jnp.logical_not(kept_valid)
    value = jnp.where(kept_valid, ts, _NEG)
    af = kept_valid.astype(f32)
    bf = other_real.astype(f32)
    n_a = jnp.sum(af)
    linc = (rows_tt <= cols_tt).astype(f32)
    rowr = lax.broadcasted_iota(jnp.int32, (_T, 1), 0).astype(f32)
    vb = tb

    out = jnp.zeros((_T, 8), f32)
    out_s = jnp.zeros((_T, 1), f32)
    off_a = jnp.float32(0.0)
    off_b = jnp.float32(0.0)
    for c in range(_NT):
        s = c * _T
        a_c = af[:, s:s + _T]
        b_c = bf[:, s:s + _T]
        cum_a = jnp.dot(a_c, linc, preferred_element_type=f32) + off_a
        cum_b = jnp.dot(b_c, linc, preferred_element_type=f32) + off_b
        rank = jnp.where(kept_valid[:, s:s + _T], cum_a - 1.0, n_a + cum_b - 1.0)
        rank = jnp.where(real[:, s:s + _T], rank, 1e6)
        sel = rank == rowr
        out = out + jnp.dot(sel.astype(f32), vb[s:s + _T, :], preferred_element_type=f32)
        out_s = out_s + jnp.sum(jnp.where(sel, value[:, s:s + _T], 0.0),
                                axis=1, keepdims=True)
        off_a = off_a + jnp.sum(a_c)
        off_b = off_b + jnp.sum(b_c)

    colid = lax.broadcasted_iota(jnp.int32, (_T, 8), 1)
    out_ref[...] = jnp.where(colid == 4, jnp.broadcast_to(out_s, (_T, 8)), out)


def kernel(boxes, scores):
    f32 = jnp.float32
    masked = jnp.where(scores > _SCORE_THRESH, scores, _NEG)
    ts, ti = lax.top_k(masked, _K)
    pad = _KP - _K
    ts_p = jnp.concatenate([ts, jnp.full((pad,), _NEG, f32)]).reshape(1, _KP)
    table = jnp.pad(boxes, ((0, 480), (0, 0))).reshape(-1)
    idx_p = jnp.concatenate([ti, jnp.full((pad,), _N, jnp.int32)])
    (t3,) = _sc_gather(table, idx_p)
    tbt8 = jnp.pad(t3.reshape(32, 4, _BW).transpose(1, 0, 2).reshape(4, _KP),
                   ((0, 4), (0, 0)))
    out8 = pl.pallas_call(
        _nms_body,
        out_shape=jax.ShapeDtypeStruct((_T, 8), f32),
        scratch_shapes=[pltpu.VMEM((1, _TN), f32)],
        interpret=_INTERPRET,
    )(ts_p, tbt8.T, tbt8)
    return out8[:_DET, :5]


# speedup vs baseline: 1.2970x; 1.2970x over previous
import jax
import jax.numpy as jnp
from jax import lax
from jax.experimental import pallas as pl
from jax.experimental.pallas import tpu as pltpu

_N = 20000
_K = 2000
_KP = 2048
_T = 128
_NT = _KP // _T
_TN = 512
_SCORE_THRESH = 0.05
_NMS_THRESH = 0.5
_DET = 100
_NEG = -1e9

_INTERPRET = False


def _nms_body(ts_ref, tb_ref, tbt_ref, out_ref, kscr_ref):
    f32 = jnp.float32
    tb = tb_ref[...]
    tbt = tbt_ref[...]
    x1c = tb[:, 0:1]
    y1c = tb[:, 1:2]
    x2c = tb[:, 2:3]
    y2c = tb[:, 3:4]
    x1r = tbt[0:1, :]
    y1r = tbt[1:2, :]
    x2r = tbt[2:3, :]
    y2r = tbt[3:4, :]
    ts = ts_ref[...]
    area_r = (x2r - x1r) * (y2r - y1r)
    area_c = (x2c - x1c) * (y2c - y1c)

    rows_tt = lax.broadcasted_iota(jnp.int32, (_T, _T), 0)
    cols_tt = lax.broadcasted_iota(jnp.int32, (_T, _T), 1)
    rows_nn = lax.broadcasted_iota(jnp.int32, (_TN, _TN), 0)
    cols_nn = lax.broadcasted_iota(jnp.int32, (_TN, _TN), 1)
    upper_f = (rows_nn < cols_nn).astype(f32)

    sup = jnp.zeros((1, _KP), jnp.bool_)
    keep_tiles = []
    for t in range(_KP // _TN):
        s = t * _TN
        tx1 = x1c[s:s + _TN]
        ty1 = y1c[s:s + _TN]
        tx2 = x2c[s:s + _TN]
        ty2 = y2c[s:s + _TN]
        ta = area_c[s:s + _TN]
        iw = jnp.clip(jnp.minimum(tx2, x2r[:, s:]) - jnp.maximum(tx1, x1r[:, s:]), 0.0)
        ih = jnp.clip(jnp.minimum(ty2, y2r[:, s:]) - jnp.maximum(ty1, y1r[:, s:]), 0.0)
        inter = iw * ih
        union = (ta + area_r[:, s:]) - inter + 1e-9
        mf = (inter > _NMS_THRESH * union).astype(f32)
        mtt = mf[:, :_TN] * upper_f
        sup_t = sup[:, s:s + _TN]
        availf = jnp.where(sup_t, 0.0, 1.0)
        kscr_ref[...] = availf

        def fbody(_, mtt=mtt, availf=availf):
            k = kscr_ref[...]
            s1 = jnp.dot(k, mtt, preferred_element_type=f32)
            k1 = availf * jnp.where(s1 < 0.5, 1.0, 0.0)
            s2 = jnp.dot(k1, mtt, preferred_element_type=f32)
            k2 = availf * jnp.where(s2 < 0.5, 1.0, 0.0)
            kscr_ref[...] = k2
            return jnp.any(k2 != k1)

        lax.while_loop(lambda c: c, fbody, jnp.bool_(True))
        kfin = kscr_ref[...] > 0.5
        keep_tiles.append(kfin)
        supnew = jnp.dot(kfin.astype(f32), mf, preferred_element_type=f32) > 0.5
        tail = sup[:, s:] | supnew
        sup = tail if s == 0 else jnp.concatenate([sup[:, :s], tail], axis=1)
    keep = jnp.concatenate(keep_tiles, axis=1)

    idx = lax.broadcasted_iota(jnp.int32, (1, _KP), 1)
    real = idx < _K
    kept_valid = keep & (ts > _NEG) & real
    other_real = real & jnp.logical_not(kept_valid)
    value = jnp.where(kept_valid, ts, _NEG)
    af = kept_valid.astype(f32)
    bf = other_real.astype(f32)
    n_a = jnp.sum(af)
    linc = (rows_tt <= cols_tt).astype(f32)
    rowr = lax.broadcasted_iota(jnp.int32, (_T, 1), 0).astype(f32)
    vb = tb

    out = jnp.zeros((_T, 8), f32)
    out_s = jnp.zeros((_T, 1), f32)
    off_a = jnp.float32(0.0)
    off_b = jnp.float32(0.0)
    for c in range(_NT):
        s = c * _T
        a_c = af[:, s:s + _T]
        b_c = bf[:, s:s + _T]
        cum_a = jnp.dot(a_c, linc, preferred_element_type=f32) + off_a
        cum_b = jnp.dot(b_c, linc, preferred_element_type=f32) + off_b
        rank = jnp.where(kept_valid[:, s:s + _T], cum_a - 1.0, n_a + cum_b - 1.0)
        rank = jnp.where(real[:, s:s + _T], rank, 1e6)
        sel = rank == rowr
        out = out + jnp.dot(sel.astype(f32), vb[s:s + _T, :], preferred_element_type=f32)
        out_s = out_s + jnp.sum(jnp.where(sel, value[:, s:s + _T], 0.0),
                                axis=1, keepdims=True)
        off_a = off_a + jnp.sum(a_c)
        off_b = off_b + jnp.sum(b_c)

    colid = lax.broadcasted_iota(jnp.int32, (_T, 8), 1)
    out_ref[...] = jnp.where(colid == 4, jnp.broadcast_to(out_s, (_T, 8)), out)


def kernel(boxes, scores):
    f32 = jnp.float32
    masked = jnp.where(scores > _SCORE_THRESH, scores, _NEG)
    ts, ti = lax.top_k(masked, _K)
    tb = jnp.take(boxes, ti, axis=0)
    pad = _KP - _K
    ts_p = jnp.concatenate([ts, jnp.full((pad,), _NEG, f32)]).reshape(1, _KP)
    tb_p = lax.dynamic_update_slice(jnp.zeros((_KP, 8), f32), tb, (0, 0))
    out8 = pl.pallas_call(
        _nms_body,
        out_shape=jax.ShapeDtypeStruct((_T, 8), f32),
        scratch_shapes=[pltpu.VMEM((1, _TN), f32)],
        interpret=_INTERPRET,
    )(ts_p, tb_p, tb_p.T)
    return out8[:_DET, :5]
